# baseline (speedup 1.0000x reference)
import functools

import jax
import jax.numpy as jnp
from jax import lax
from jax.experimental import pallas as pl
from jax.experimental.pallas import tpu as pltpu
from jax.experimental.pallas import tpu_sc as plsc

D = 2048
N_EXP = 64
TOP_K = 8
N_TOK = 16384

TB = 2048

NC, NS, L = 2, 16, 16
NW = NC * NS
T_PER_W = N_TOK // NW
CHUNK = L
N_CHUNK = T_PER_W // CHUNK


def _gates_block(x_ref, w_ref, gates_ref):
    logits = jax.lax.dot_general(
        x_ref[...], w_ref[...], (((1,), (1,)), ((), ())),
        preferred_element_type=jnp.float32,
    )
    lt = logits.T
    m = jnp.max(lt, axis=0, keepdims=True)
    e = jnp.exp(lt - m)
    s = jnp.sum(e, axis=0, keepdims=True)
    gates_ref[...] = (e / s).T


def _tc_gates(hidden_states, W_gate):
    grid = (N_TOK // TB,)
    return pl.pallas_call(
        _gates_block,
        grid=grid,
        in_specs=[
            pl.BlockSpec((TB, D), lambda i: (i, 0)),
            pl.BlockSpec((N_EXP, D), lambda i: (0, 0)),
        ],
        out_specs=pl.BlockSpec((TB, N_EXP), lambda i: (i, 0)),
        out_shape=jax.ShapeDtypeStruct((N_TOK, N_EXP), jnp.float32),
        compiler_params=pltpu.CompilerParams(
            dimension_semantics=("parallel",),
        ),
    )(hidden_states, W_gate)


@functools.partial(
    pl.kernel,
    out_type=[
        jax.ShapeDtypeStruct((N_TOK, TOP_K), jnp.float32),
        jax.ShapeDtypeStruct((N_TOK, TOP_K), jnp.int32),
    ],
    mesh=plsc.VectorSubcoreMesh(core_axis_name="c", subcore_axis_name="s"),
    compiler_params=pltpu.CompilerParams(needs_layout_passes=False),
    scratch_types=[
        pltpu.VMEM((CHUNK, N_EXP), jnp.float32),
        pltpu.VMEM((N_EXP * L,), jnp.float32),
        pltpu.VMEM((CHUNK, TOP_K), jnp.float32),
        pltpu.VMEM((CHUNK, TOP_K), jnp.int32),
    ],
)
def _sc_topk(gates_hbm, vals_hbm, inds_hbm, tile_v, te_v, vals_v, inds_v):
    wid = lax.axis_index("s") * NC + lax.axis_index("c")
    lanes = lax.broadcasted_iota(jnp.int32, (L,), 0)

    def chunk_body(c, _):
        tok = wid * T_PER_W + c * CHUNK
        pltpu.sync_copy(gates_hbm.at[pl.ds(tok, CHUNK)], tile_v)

        def tr_body(e, _):
            col = plsc.load_gather(tile_v, [lanes, jnp.full((L,), e, jnp.int32)])
            te_v[pl.ds(e * L, L)] = col
            return 0

        lax.fori_loop(0, N_EXP, tr_body, 0)

        for j in range(TOP_K):
            def sel_body(e, carry):
                m, am = carry
                v = te_v[pl.ds(e * L, L)]
                gt = v > m
                return (
                    jnp.where(gt, v, m),
                    jnp.where(gt, jnp.full((L,), e, jnp.int32), am),
                )

            m, am = lax.fori_loop(
                0, N_EXP, sel_body,
                (jnp.full((L,), -1.0, jnp.float32),
                 jnp.zeros((L,), jnp.int32)),
            )
            plsc.store_scatter(
                vals_v, [lanes, jnp.full((L,), j, jnp.int32)], m
            )
            plsc.store_scatter(
                inds_v, [lanes, jnp.full((L,), j, jnp.int32)], am
            )
            plsc.store_scatter(
                te_v, [am * L + lanes], jnp.full((L,), -1.0, jnp.float32)
            )

        pltpu.sync_copy(vals_v, vals_hbm.at[pl.ds(tok, CHUNK)])
        pltpu.sync_copy(inds_v, inds_hbm.at[pl.ds(tok, CHUNK)])
        return 0

    lax.fori_loop(0, N_CHUNK, chunk_body, 0)


@jax.jit
def kernel(hidden_states, W_gate, W_noise):
    del W_noise
    gates = _tc_gates(hidden_states, W_gate)
    vals, inds = _sc_topk(gates)
    return vals, inds, gates


# speedup vs baseline: 2.9644x; 2.9644x over previous
import functools

import jax
import jax.numpy as jnp
from jax.experimental import pallas as pl
from jax.experimental.pallas import tpu as pltpu

D = 2048
N_EXP = 64
TOP_K = 8
N_TOK = 16384

TB = 2048


def _router_block(x_ref, w_ref, vals_ref, inds_ref, gates_ref):
    x = x_ref[...]
    w = w_ref[...]
    logits = jax.lax.dot_general(
        x, w, (((1,), (1,)), ((), ())), preferred_element_type=jnp.float32
    )
    lt = logits.T
    m = jnp.max(lt, axis=0, keepdims=True)
    e = jnp.exp(lt - m)
    s = jnp.sum(e, axis=0, keepdims=True)
    gt = e / s
    gates_ref[...] = gt.T

    iota = jax.lax.broadcasted_iota(jnp.int32, (N_EXP, TB), 0)
    work = gt
    vals_rows = []
    inds_rows = []
    for _ in range(TOP_K):
        mx = jnp.max(work, axis=0, keepdims=True)
        idx = jnp.min(jnp.where(work == mx, iota, N_EXP), axis=0, keepdims=True)
        vals_rows.append(mx)
        inds_rows.append(idx)
        work = jnp.where(iota == idx, -1.0, work)
    vals_ref[...] = jnp.concatenate(vals_rows, axis=0).T
    inds_ref[...] = jnp.concatenate(inds_rows, axis=0).T


@jax.jit
def kernel(hidden_states, W_gate, W_noise):
    del W_noise
    grid = (N_TOK // TB,)
    vals, inds, gates = pl.pallas_call(
        _router_block,
        grid=grid,
        in_specs=[
            pl.BlockSpec((TB, D), lambda i: (i, 0)),
            pl.BlockSpec((N_EXP, D), lambda i: (0, 0)),
        ],
        out_specs=[
            pl.BlockSpec((TB, TOP_K), lambda i: (i, 0)),
            pl.BlockSpec((TB, TOP_K), lambda i: (i, 0)),
            pl.BlockSpec((TB, N_EXP), lambda i: (i, 0)),
        ],
        out_shape=[
            jax.ShapeDtypeStruct((N_TOK, TOP_K), jnp.float32),
            jax.ShapeDtypeStruct((N_TOK, TOP_K), jnp.int32),
            jax.ShapeDtypeStruct((N_TOK, N_EXP), jnp.float32),
        ],
        compiler_params=pltpu.CompilerParams(
            dimension_semantics=("parallel",),
        ),
    )(hidden_states, W_gate)
    return vals, inds, gates
